# baseline (speedup 1.0000x reference)
import numpy as np
import jax
import jax.numpy as jnp
from jax.experimental import pallas as pl
from jax.experimental.pallas import tpu as pltpu

_TB = 128
_BIG = 1e30


def _body(x_ref, w1_ref, t1_ref, w2_ref, t2_ref, wf_ref, bf_ref,
          wt_ref, bt_ref, o_ref):
    tb = _TB
    rq = tb * 10
    bb = jnp.concatenate([x_ref[...], jnp.zeros((8, 144), jnp.float32)], axis=0)

    def conv1(lhs):
        y = jnp.dot(lhs, w1_ref[...], preferred_element_type=jnp.float32)
        k = jnp.where(y > t1_ref[...], 1.0, 0.0)
        return k[:, 0:256] + k[:, 256:512]

    wp0 = conv1(bb[0:rq, 36:144])
    wp1 = conv1(jnp.concatenate([bb[0:rq, 72:144], bb[1:rq + 1, 0:36]], axis=1))
    wp2 = conv1(jnp.concatenate([bb[0:rq, 108:144], bb[1:rq + 1, 0:72]], axis=1))
    wp3 = conv1(bb[1:rq + 1, 0:108])
    hpe = wp0 + wp1
    hpo = wp2 + wp3
    q = jax.lax.broadcasted_iota(jnp.int32, (rq, 1), 0)
    ze = jnp.where(q % 10 != 0, hpe, 0.0)
    zo = jnp.where(q % 10 != 9, hpo, 0.0)

    m = rq - 1
    ze1 = ze[1:rq]
    zo1 = zo[1:rq]
    wd0 = w2_ref[0:256, :]
    wd1 = w2_ref[256:512, :]
    wd2 = w2_ref[512:768, :]

    def conv2(p0, p1, p2):
        y = jnp.dot(p0, wd0, preferred_element_type=jnp.float32)
        y = y + jnp.dot(p1, wd1, preferred_element_type=jnp.float32)
        y = y + jnp.dot(p2, wd2, preferred_element_type=jnp.float32)
        k = jnp.where(y > t2_ref[...], 1.0, 0.0)
        return k[:, 0:128] + k[:, 128:256]

    wpe = conv2(ze[0:m], zo[0:m], ze1)
    wpo = conv2(zo[0:m], ze1, zo1)
    hp2 = jnp.pad(wpe + wpo, ((0, 1), (0, 0)))

    flat = hp2.reshape(tb, 1280)
    h = jnp.dot(flat, wf_ref[...], preferred_element_type=jnp.float32) + bf_ref[...]
    o_ref[...] = jnp.dot(h, wt_ref[...], preferred_element_type=jnp.float32) + bt_ref[...]


def _band(win, wout):
    u = np.arange(win)[:, None, None, None]
    pw = np.arange(2)[None, :, None, None]
    w2 = np.arange(wout)[None, None, :, None]
    dx = np.arange(3)[None, None, None, :]
    return jnp.asarray((u == 2 * w2 + pw + dx - 1).astype(np.float32))


def _padlane(a, n):
    return jnp.pad(a, [(0, 0)] * (a.ndim - 1) + [(0, n - a.shape[-1])])


def kernel(conv1_w, conv1_b, conv2_w, conv2_b, fc_w, fc_b, task_w, task_b, x):
    b = x.shape[0]
    n_tasks = task_w.shape[0]
    tb = _TB
    assert b % tb == 0, b

    e1 = _band(36, 18)
    w1 = jnp.einsum('upwx,cdx->dupcw', e1, conv1_w[:, 0])
    w1 = _padlane(w1.reshape(108, 2, 180), 256).reshape(108, 512)
    t1 = _padlane(jnp.tile(jnp.repeat(0.5 - conv1_b, 18), 2).reshape(2, 180), 256)
    t1 = (t1 + _BIG * (jnp.arange(256) >= 180)).reshape(1, 512)
    e2 = _band(18, 9)
    w2 = 0.25 * jnp.einsum('upwx,cidx->diupcw', e2, conv2_w)
    w2 = _padlane(w2.reshape(3, 180, 2, 90), 128)
    w2 = jnp.pad(w2, ((0, 0), (0, 76), (0, 0), (0, 0))).reshape(768, 256)
    t2 = _padlane(jnp.tile(jnp.repeat(0.5 - conv2_b, 9), 2).reshape(2, 90), 128)
    t2 = (t2 + _BIG * (jnp.arange(128) >= 90)).reshape(1, 256)
    hh, cc_, ww = np.meshgrid(np.arange(9), np.arange(10), np.arange(9), indexing='ij')
    perm = (cc_ * 81 + hh * 9 + ww).reshape(-1)
    wf = jnp.concatenate([0.25 * fc_w.T[perm, :], jnp.zeros((90, 50), jnp.float32)])
    wf = jnp.pad(wf.reshape(10, 90, 50), ((0, 0), (0, 38), (0, 0))).reshape(1280, 50)
    bf = fc_b.reshape(1, 50)
    wt = jnp.transpose(task_w, (2, 0, 1)).reshape(50, n_tasks * 10)
    bt = task_b.reshape(1, n_tasks * 10)

    xs = jnp.pad(x.reshape(b, 9, 144), ((0, 0), (1, 0), (0, 0))).reshape(b * 10, 144)

    out = pl.pallas_call(
        _body,
        out_shape=jax.ShapeDtypeStruct((b, n_tasks * 10), jnp.float32),
        grid=(b // tb,),
        in_specs=[
            pl.BlockSpec((tb * 10, 144), lambda i: (i, 0)),
            pl.BlockSpec((108, 512), lambda i: (0, 0)),
            pl.BlockSpec((1, 512), lambda i: (0, 0)),
            pl.BlockSpec((768, 256), lambda i: (0, 0)),
            pl.BlockSpec((1, 256), lambda i: (0, 0)),
            pl.BlockSpec((1280, 50), lambda i: (0, 0)),
            pl.BlockSpec((1, 50), lambda i: (0, 0)),
            pl.BlockSpec((50, n_tasks * 10), lambda i: (0, 0)),
            pl.BlockSpec((1, n_tasks * 10), lambda i: (0, 0)),
        ],
        out_specs=pl.BlockSpec((tb, n_tasks * 10), lambda i: (i, 0)),
        compiler_params=pltpu.CompilerParams(dimension_semantics=("parallel",)),
    )(xs, w1, t1, w2, t2, wf, bf, wt, bt)
    return out.reshape(b, n_tasks, 10)


# speedup vs baseline: 1.3025x; 1.3025x over previous
import numpy as np
import jax
import jax.numpy as jnp
from jax.experimental import pallas as pl
from jax.experimental.pallas import tpu as pltpu

_TB = 128
_BIG = 1e30


def _body(x_ref, w1_ref, t1_ref, w2_ref, t2_ref, wf_ref, bf_ref,
          wt_ref, bt_ref, o_ref, b_ref):
    tb = _TB
    rq = tb * 10
    b_ref[...] = jnp.zeros(b_ref.shape, jnp.float32)
    for im in range(tb):
        b_ref[im * 10 + 1: im * 10 + 10, :] = x_ref[im * 9: im * 9 + 9, :]
    bb = b_ref[...]

    def conv1(lhs):
        y = jnp.dot(lhs, w1_ref[...], preferred_element_type=jnp.float32)
        k = jnp.where(y > t1_ref[...], 1.0, 0.0)
        return k[:, 0:256] + k[:, 256:512]

    wp0 = conv1(bb[0:rq, 36:144])
    wp1 = conv1(jnp.concatenate([bb[0:rq, 72:144], bb[1:rq + 1, 0:36]], axis=1))
    wp2 = conv1(jnp.concatenate([bb[0:rq, 108:144], bb[1:rq + 1, 0:72]], axis=1))
    wp3 = conv1(bb[1:rq + 1, 0:108])
    hpe = wp0 + wp1
    hpo = wp2 + wp3
    q = jax.lax.broadcasted_iota(jnp.int32, (rq, 1), 0)
    ze = jnp.where(q % 10 != 0, hpe, 0.0)
    zo = jnp.where(q % 10 != 9, hpo, 0.0)

    m = rq - 1
    ze1 = ze[1:rq]
    zo1 = zo[1:rq]
    wd0 = w2_ref[0:256, :]
    wd1 = w2_ref[256:512, :]
    wd2 = w2_ref[512:768, :]

    def conv2(p0, p1, p2):
        y = jnp.dot(p0, wd0, preferred_element_type=jnp.float32)
        y = y + jnp.dot(p1, wd1, preferred_element_type=jnp.float32)
        y = y + jnp.dot(p2, wd2, preferred_element_type=jnp.float32)
        k = jnp.where(y > t2_ref[...], 1.0, 0.0)
        return k[:, 0:128] + k[:, 128:256]

    wpe = conv2(ze[0:m], zo[0:m], ze1)
    wpo = conv2(zo[0:m], ze1, zo1)
    hp2 = jnp.pad(wpe + wpo, ((0, 1), (0, 0)))

    flat = hp2.reshape(tb, 1280)
    h = jnp.dot(flat, wf_ref[...], preferred_element_type=jnp.float32) + bf_ref[...]
    o_ref[...] = jnp.dot(h, wt_ref[...], preferred_element_type=jnp.float32) + bt_ref[...]


def _band(win, wout):
    u = np.arange(win)[:, None, None, None]
    pw = np.arange(2)[None, :, None, None]
    w2 = np.arange(wout)[None, None, :, None]
    dx = np.arange(3)[None, None, None, :]
    return jnp.asarray((u == 2 * w2 + pw + dx - 1).astype(np.float32))


def _padlane(a, n):
    return jnp.pad(a, [(0, 0)] * (a.ndim - 1) + [(0, n - a.shape[-1])])


def kernel(conv1_w, conv1_b, conv2_w, conv2_b, fc_w, fc_b, task_w, task_b, x):
    b = x.shape[0]
    n_tasks = task_w.shape[0]
    tb = _TB
    assert b % tb == 0, b

    e1 = _band(36, 18)
    w1 = jnp.einsum('upwx,cdx->dupcw', e1, conv1_w[:, 0])
    w1 = _padlane(w1.reshape(108, 2, 180), 256).reshape(108, 512)
    t1 = _padlane(jnp.tile(jnp.repeat(0.5 - conv1_b, 18), 2).reshape(2, 180), 256)
    t1 = (t1 + _BIG * (jnp.arange(256) >= 180)).reshape(1, 512)
    e2 = _band(18, 9)
    w2 = 0.25 * jnp.einsum('upwx,cidx->diupcw', e2, conv2_w)
    w2 = _padlane(w2.reshape(3, 180, 2, 90), 128)
    w2 = jnp.pad(w2, ((0, 0), (0, 76), (0, 0), (0, 0))).reshape(768, 256)
    t2 = _padlane(jnp.tile(jnp.repeat(0.5 - conv2_b, 9), 2).reshape(2, 90), 128)
    t2 = (t2 + _BIG * (jnp.arange(128) >= 90)).reshape(1, 256)
    hh, cc_, ww = np.meshgrid(np.arange(9), np.arange(10), np.arange(9), indexing='ij')
    perm = (cc_ * 81 + hh * 9 + ww).reshape(-1)
    wf = jnp.concatenate([0.25 * fc_w.T[perm, :], jnp.zeros((90, 50), jnp.float32)])
    wf = jnp.pad(wf.reshape(10, 90, 50), ((0, 0), (0, 38), (0, 0))).reshape(1280, 50)
    bf = fc_b.reshape(1, 50)
    wt = jnp.transpose(task_w, (2, 0, 1)).reshape(50, n_tasks * 10)
    bt = task_b.reshape(1, n_tasks * 10)

    xs = x.reshape(b * 9, 144)

    out = pl.pallas_call(
        _body,
        out_shape=jax.ShapeDtypeStruct((b, n_tasks * 10), jnp.float32),
        grid=(b // tb,),
        in_specs=[
            pl.BlockSpec((tb * 9, 144), lambda i: (i, 0)),
            pl.BlockSpec((108, 512), lambda i: (0, 0)),
            pl.BlockSpec((1, 512), lambda i: (0, 0)),
            pl.BlockSpec((768, 256), lambda i: (0, 0)),
            pl.BlockSpec((1, 256), lambda i: (0, 0)),
            pl.BlockSpec((1280, 50), lambda i: (0, 0)),
            pl.BlockSpec((1, 50), lambda i: (0, 0)),
            pl.BlockSpec((50, n_tasks * 10), lambda i: (0, 0)),
            pl.BlockSpec((1, n_tasks * 10), lambda i: (0, 0)),
        ],
        out_specs=pl.BlockSpec((tb, n_tasks * 10), lambda i: (i, 0)),
        scratch_shapes=[pltpu.VMEM((tb * 10 + 8, 144), jnp.float32)],
        compiler_params=pltpu.CompilerParams(dimension_semantics=("parallel",)),
    )(xs, w1, t1, w2, t2, wf, bf, wt, bt)
    return out.reshape(b, n_tasks, 10)


# speedup vs baseline: 1.3087x; 1.0048x over previous
import numpy as np
import jax
import jax.numpy as jnp
from jax.experimental import pallas as pl
from jax.experimental.pallas import tpu as pltpu

_TB = 128
_BIG = 1e30


def _body(x_ref, w1_ref, t1_ref, w2_ref, t2_ref, wf_ref, bf_ref,
          wt_ref, bt_ref, o_ref, b_ref):
    tb = _TB
    rq = tb * 10
    for im in range(tb):
        b_ref[im * 10: im * 10 + 1, :] = jnp.zeros((1, 144), jnp.float32)
        b_ref[im * 10 + 1: im * 10 + 10, :] = x_ref[im * 9: im * 9 + 9, :]
    b_ref[tb * 10: tb * 10 + 8, :] = jnp.zeros((8, 144), jnp.float32)
    bb = b_ref[...]

    def conv1(lhs):
        y = jnp.dot(lhs, w1_ref[...], preferred_element_type=jnp.float32)
        k = jnp.where(y > t1_ref[...], 1.0, 0.0)
        return k[:, 0:256] + k[:, 256:512]

    wp0 = conv1(bb[0:rq, 36:144])
    wp1 = conv1(jnp.concatenate([bb[0:rq, 72:144], bb[1:rq + 1, 0:36]], axis=1))
    wp2 = conv1(jnp.concatenate([bb[0:rq, 108:144], bb[1:rq + 1, 0:72]], axis=1))
    wp3 = conv1(bb[1:rq + 1, 0:108])
    hpe = wp0 + wp1
    hpo = wp2 + wp3
    q = jax.lax.broadcasted_iota(jnp.int32, (rq, 1), 0)
    ze = jnp.where(q % 10 != 0, hpe, 0.0)
    zo = jnp.where(q % 10 != 9, hpo, 0.0)

    ze1 = pltpu.roll(ze, rq - 1, 0)
    zo1 = pltpu.roll(zo, rq - 1, 0)
    wd0 = w2_ref[0:256, :]
    wd1 = w2_ref[256:512, :]
    wd2 = w2_ref[512:768, :]

    def conv2(p0, p1, p2):
        y = jnp.dot(p0, wd0, preferred_element_type=jnp.float32)
        y = y + jnp.dot(p1, wd1, preferred_element_type=jnp.float32)
        y = y + jnp.dot(p2, wd2, preferred_element_type=jnp.float32)
        k = jnp.where(y > t2_ref[...], 1.0, 0.0)
        return k[:, 0:128] + k[:, 128:256]

    wpe = conv2(ze, zo, ze1)
    wpo = conv2(zo, ze1, zo1)
    hp2 = wpe + wpo

    flat = hp2.reshape(tb, 1280)
    h = jnp.dot(flat, wf_ref[...], preferred_element_type=jnp.float32) + bf_ref[...]
    o_ref[...] = jnp.dot(h, wt_ref[...], preferred_element_type=jnp.float32) + bt_ref[...]


def _band(win, wout):
    u = np.arange(win)[:, None, None, None]
    pw = np.arange(2)[None, :, None, None]
    w2 = np.arange(wout)[None, None, :, None]
    dx = np.arange(3)[None, None, None, :]
    return jnp.asarray((u == 2 * w2 + pw + dx - 1).astype(np.float32))


def _padlane(a, n):
    return jnp.pad(a, [(0, 0)] * (a.ndim - 1) + [(0, n - a.shape[-1])])


def kernel(conv1_w, conv1_b, conv2_w, conv2_b, fc_w, fc_b, task_w, task_b, x):
    b = x.shape[0]
    n_tasks = task_w.shape[0]
    tb = _TB
    assert b % tb == 0, b

    e1 = _band(36, 18)
    w1 = jnp.einsum('upwx,cdx->dupcw', e1, conv1_w[:, 0])
    w1 = _padlane(w1.reshape(108, 2, 180), 256).reshape(108, 512)
    t1 = _padlane(jnp.tile(jnp.repeat(0.5 - conv1_b, 18), 2).reshape(2, 180), 256)
    t1 = (t1 + _BIG * (jnp.arange(256) >= 180)).reshape(1, 512)
    e2 = _band(18, 9)
    w2 = 0.25 * jnp.einsum('upwx,cidx->diupcw', e2, conv2_w)
    w2 = _padlane(w2.reshape(3, 180, 2, 90), 128)
    w2 = jnp.pad(w2, ((0, 0), (0, 76), (0, 0), (0, 0))).reshape(768, 256)
    t2 = _padlane(jnp.tile(jnp.repeat(0.5 - conv2_b, 9), 2).reshape(2, 90), 128)
    t2 = (t2 + _BIG * (jnp.arange(128) >= 90)).reshape(1, 256)
    hh, cc_, ww = np.meshgrid(np.arange(9), np.arange(10), np.arange(9), indexing='ij')
    perm = (cc_ * 81 + hh * 9 + ww).reshape(-1)
    wf = jnp.concatenate([0.25 * fc_w.T[perm, :], jnp.zeros((90, 50), jnp.float32)])
    wf = jnp.pad(wf.reshape(10, 90, 50), ((0, 0), (0, 38), (0, 0))).reshape(1280, 50)
    bf = fc_b.reshape(1, 50)
    wt = jnp.transpose(task_w, (2, 0, 1)).reshape(50, n_tasks * 10)
    bt = task_b.reshape(1, n_tasks * 10)

    xs = x.reshape(b * 9, 144)

    out = pl.pallas_call(
        _body,
        out_shape=jax.ShapeDtypeStruct((b, n_tasks * 10), jnp.float32),
        grid=(b // tb,),
        in_specs=[
            pl.BlockSpec((tb * 9, 144), lambda i: (i, 0)),
            pl.BlockSpec((108, 512), lambda i: (0, 0)),
            pl.BlockSpec((1, 512), lambda i: (0, 0)),
            pl.BlockSpec((768, 256), lambda i: (0, 0)),
            pl.BlockSpec((1, 256), lambda i: (0, 0)),
            pl.BlockSpec((1280, 50), lambda i: (0, 0)),
            pl.BlockSpec((1, 50), lambda i: (0, 0)),
            pl.BlockSpec((50, n_tasks * 10), lambda i: (0, 0)),
            pl.BlockSpec((1, n_tasks * 10), lambda i: (0, 0)),
        ],
        out_specs=pl.BlockSpec((tb, n_tasks * 10), lambda i: (i, 0)),
        scratch_shapes=[pltpu.VMEM((tb * 10 + 8, 144), jnp.float32)],
        compiler_params=pltpu.CompilerParams(dimension_semantics=("parallel",)),
    )(xs, w1, t1, w2, t2, wf, bf, wt, bt)
    return out.reshape(b, n_tasks, 10)


# speedup vs baseline: 1.3302x; 1.0164x over previous
import numpy as np
import jax
import jax.numpy as jnp
from jax.experimental import pallas as pl
from jax.experimental.pallas import tpu as pltpu

_TB = 256
_BIG = 1e30


def _body(x_ref, w1_ref, t1_ref, w2_ref, t2_ref, wf_ref, bf_ref,
          wt_ref, bt_ref, o_ref, b_ref):
    tb = _TB
    rq = tb * 10
    for im in range(tb):
        b_ref[im * 10: im * 10 + 1, :] = jnp.zeros((1, 144), jnp.float32)
        b_ref[im * 10 + 1: im * 10 + 10, :] = x_ref[im * 9: im * 9 + 9, :]
    b_ref[tb * 10: tb * 10 + 8, :] = jnp.zeros((8, 144), jnp.float32)
    bb = b_ref[...]

    def conv1(lhs):
        y = jnp.dot(lhs, w1_ref[...], preferred_element_type=jnp.float32)
        k = jnp.where(y > t1_ref[...], 1.0, 0.0)
        return k[:, 0:256] + k[:, 256:512]

    wp0 = conv1(bb[0:rq, 36:144])
    wp1 = conv1(jnp.concatenate([bb[0:rq, 72:144], bb[1:rq + 1, 0:36]], axis=1))
    wp2 = conv1(jnp.concatenate([bb[0:rq, 108:144], bb[1:rq + 1, 0:72]], axis=1))
    wp3 = conv1(bb[1:rq + 1, 0:108])
    hpe = wp0 + wp1
    hpo = wp2 + wp3
    q = jax.lax.broadcasted_iota(jnp.int32, (rq, 1), 0)
    ze = jnp.where(q % 10 != 0, hpe, 0.0)
    zo = jnp.where(q % 10 != 9, hpo, 0.0)

    ze1 = pltpu.roll(ze, rq - 1, 0)
    zo1 = pltpu.roll(zo, rq - 1, 0)
    wd0 = w2_ref[0:256, :]
    wd1 = w2_ref[256:512, :]
    wd2 = w2_ref[512:768, :]

    def conv2(p0, p1, p2):
        y = jnp.dot(p0, wd0, preferred_element_type=jnp.float32)
        y = y + jnp.dot(p1, wd1, preferred_element_type=jnp.float32)
        y = y + jnp.dot(p2, wd2, preferred_element_type=jnp.float32)
        k = jnp.where(y > t2_ref[...], 1.0, 0.0)
        return k[:, 0:128] + k[:, 128:256]

    wpe = conv2(ze, zo, ze1)
    wpo = conv2(zo, ze1, zo1)
    hp2 = wpe + wpo

    flat = hp2.reshape(tb, 1280)
    h = jnp.dot(flat, wf_ref[...], preferred_element_type=jnp.float32) + bf_ref[...]
    o_ref[...] = jnp.dot(h, wt_ref[...], preferred_element_type=jnp.float32) + bt_ref[...]


def _band(win, wout):
    u = np.arange(win)[:, None, None, None]
    pw = np.arange(2)[None, :, None, None]
    w2 = np.arange(wout)[None, None, :, None]
    dx = np.arange(3)[None, None, None, :]
    return jnp.asarray((u == 2 * w2 + pw + dx - 1).astype(np.float32))


def _padlane(a, n):
    return jnp.pad(a, [(0, 0)] * (a.ndim - 1) + [(0, n - a.shape[-1])])


def kernel(conv1_w, conv1_b, conv2_w, conv2_b, fc_w, fc_b, task_w, task_b, x):
    b = x.shape[0]
    n_tasks = task_w.shape[0]
    tb = _TB
    assert b % tb == 0, b

    e1 = _band(36, 18)
    w1 = jnp.einsum('upwx,cdx->dupcw', e1, conv1_w[:, 0])
    w1 = _padlane(w1.reshape(108, 2, 180), 256).reshape(108, 512)
    t1 = _padlane(jnp.tile(jnp.repeat(0.5 - conv1_b, 18), 2).reshape(2, 180), 256)
    t1 = (t1 + _BIG * (jnp.arange(256) >= 180)).reshape(1, 512)
    e2 = _band(18, 9)
    w2 = 0.25 * jnp.einsum('upwx,cidx->diupcw', e2, conv2_w)
    w2 = _padlane(w2.reshape(3, 180, 2, 90), 128)
    w2 = jnp.pad(w2, ((0, 0), (0, 76), (0, 0), (0, 0))).reshape(768, 256)
    t2 = _padlane(jnp.tile(jnp.repeat(0.5 - conv2_b, 9), 2).reshape(2, 90), 128)
    t2 = (t2 + _BIG * (jnp.arange(128) >= 90)).reshape(1, 256)
    hh, cc_, ww = np.meshgrid(np.arange(9), np.arange(10), np.arange(9), indexing='ij')
    perm = (cc_ * 81 + hh * 9 + ww).reshape(-1)
    wf = jnp.concatenate([0.25 * fc_w.T[perm, :], jnp.zeros((90, 50), jnp.float32)])
    wf = jnp.pad(wf.reshape(10, 90, 50), ((0, 0), (0, 38), (0, 0))).reshape(1280, 50)
    bf = fc_b.reshape(1, 50)
    wt = jnp.transpose(task_w, (2, 0, 1)).reshape(50, n_tasks * 10)
    bt = task_b.reshape(1, n_tasks * 10)

    xs = x.reshape(b * 9, 144)

    out = pl.pallas_call(
        _body,
        out_shape=jax.ShapeDtypeStruct((b, n_tasks * 10), jnp.float32),
        grid=(b // tb,),
        in_specs=[
            pl.BlockSpec((tb * 9, 144), lambda i: (i, 0)),
            pl.BlockSpec((108, 512), lambda i: (0, 0)),
            pl.BlockSpec((1, 512), lambda i: (0, 0)),
            pl.BlockSpec((768, 256), lambda i: (0, 0)),
            pl.BlockSpec((1, 256), lambda i: (0, 0)),
            pl.BlockSpec((1280, 50), lambda i: (0, 0)),
            pl.BlockSpec((1, 50), lambda i: (0, 0)),
            pl.BlockSpec((50, n_tasks * 10), lambda i: (0, 0)),
            pl.BlockSpec((1, n_tasks * 10), lambda i: (0, 0)),
        ],
        out_specs=pl.BlockSpec((tb, n_tasks * 10), lambda i: (i, 0)),
        scratch_shapes=[pltpu.VMEM((tb * 10 + 8, 144), jnp.float32)],
        compiler_params=pltpu.CompilerParams(dimension_semantics=("parallel",)),
    )(xs, w1, t1, w2, t2, wf, bf, wt, bt)
    return out.reshape(b, n_tasks, 10)


# speedup vs baseline: 1.3321x; 1.0014x over previous
import numpy as np
import jax
import jax.numpy as jnp
from jax.experimental import pallas as pl
from jax.experimental.pallas import tpu as pltpu

_TB = 256
_BIG = 1e30


def _body(x_ref, w1_ref, t1_ref, w2_ref, t2_ref, wf_ref, bf_ref,
          wt_ref, bt_ref, o_ref, b_ref):
    tb = _TB
    rq = tb * 10
    for im in range(tb):
        b_ref[im * 10: im * 10 + 1, :] = jnp.zeros((1, 144), jnp.float32)
        b_ref[im * 10 + 1: im * 10 + 10, :] = x_ref[im * 9: im * 9 + 9, :]
    b_ref[tb * 10: tb * 10 + 8, :] = jnp.zeros((8, 144), jnp.float32)
    bb = b_ref[...]

    def conv1(lhs):
        y = jnp.dot(lhs, w1_ref[...], preferred_element_type=jnp.float32)
        k = jnp.where(y > t1_ref[...], 1.0, 0.0)
        return k[:, 0:256] + k[:, 256:512]

    wp0 = conv1(bb[0:rq, 36:144])
    wp1 = conv1(jnp.concatenate([bb[0:rq, 72:144], bb[1:rq + 1, 0:36]], axis=1))
    wp2 = conv1(jnp.concatenate([bb[0:rq, 108:144], bb[1:rq + 1, 0:72]], axis=1))
    wp3 = conv1(bb[1:rq + 1, 0:108])
    hpe = wp0 + wp1
    hpo = wp2 + wp3
    q = jax.lax.broadcasted_iota(jnp.int32, (rq, 1), 0)
    ze = jnp.where(q % 10 != 0, hpe, 0.0)
    zo = jnp.where(q % 10 != 9, hpo, 0.0)

    ze1 = pltpu.roll(ze, rq - 1, 0)
    zo1 = pltpu.roll(zo, rq - 1, 0)
    wd0 = w2_ref[0:256, :]
    wd1 = w2_ref[256:512, :]
    wd2 = w2_ref[512:768, :]

    def conv2(p0, p1, p2):
        y = jnp.dot(p0, wd0, preferred_element_type=jnp.float32)
        y = y + jnp.dot(p1, wd1, preferred_element_type=jnp.float32)
        y = y + jnp.dot(p2, wd2, preferred_element_type=jnp.float32)
        k = jnp.where(y > t2_ref[...], 1.0, 0.0)
        return k[:, 0:128] + k[:, 128:256]

    wpe = conv2(ze, zo, ze1)
    wpo = conv2(zo, ze1, zo1)
    hp2 = wpe + wpo

    flat = hp2.reshape(tb, 1280)
    h = jnp.dot(flat, wf_ref[...], preferred_element_type=jnp.float32) + bf_ref[...]
    o_ref[...] = jnp.dot(h, wt_ref[...], preferred_element_type=jnp.float32) + bt_ref[...]


def _band(win, wout):
    u = np.arange(win)[:, None, None, None]
    pw = np.arange(2)[None, :, None, None]
    w2 = np.arange(wout)[None, None, :, None]
    dx = np.arange(3)[None, None, None, :]
    return jnp.asarray((u == 2 * w2 + pw + dx - 1).astype(np.float32))


def _padlane(a, n):
    return jnp.pad(a, [(0, 0)] * (a.ndim - 1) + [(0, n - a.shape[-1])])


def kernel(conv1_w, conv1_b, conv2_w, conv2_b, fc_w, fc_b, task_w, task_b, x):
    b = x.shape[0]
    n_tasks = task_w.shape[0]
    tb = _TB
    assert b % tb == 0, b

    e1 = _band(36, 18)
    w1 = jnp.einsum('upwx,cdx->dupcw', e1, conv1_w[:, 0])
    w1 = _padlane(w1.reshape(108, 2, 180), 256).reshape(108, 512)
    t1 = _padlane(jnp.tile(jnp.repeat(0.5 - conv1_b, 18), 2).reshape(2, 180), 256)
    t1 = (t1 + _BIG * (jnp.arange(256) >= 180)).reshape(1, 512)
    e2 = _band(18, 9)
    w2 = 0.25 * jnp.einsum('upwx,cidx->diupcw', e2, conv2_w)
    w2 = _padlane(w2.reshape(3, 180, 2, 90), 128)
    w2 = jnp.pad(w2, ((0, 0), (0, 76), (0, 0), (0, 0))).reshape(768, 256)
    t2 = _padlane(jnp.tile(jnp.repeat(0.5 - conv2_b, 9), 2).reshape(2, 90), 128)
    t2 = (t2 + _BIG * (jnp.arange(128) >= 90)).reshape(1, 256)
    hh, cc_, ww = np.meshgrid(np.arange(9), np.arange(10), np.arange(9), indexing='ij')
    perm = (cc_ * 81 + hh * 9 + ww).reshape(-1)
    wf = jnp.concatenate([0.25 * fc_w.T[perm, :], jnp.zeros((90, 50), jnp.float32)])
    wf = jnp.pad(wf.reshape(10, 90, 50), ((0, 0), (0, 38), (0, 0))).reshape(1280, 50)
    bf = fc_b.reshape(1, 50)
    wt = jnp.transpose(task_w, (2, 0, 1)).reshape(50, n_tasks * 10)
    bt = task_b.reshape(1, n_tasks * 10)

    xs = x.reshape(b * 9, 144)

    nh = b // tb // 2
    out = pl.pallas_call(
        _body,
        out_shape=jax.ShapeDtypeStruct((b, n_tasks * 10), jnp.float32),
        grid=(2, nh),
        in_specs=[
            pl.BlockSpec((tb * 9, 144), lambda c, i: (c * nh + i, 0)),
            pl.BlockSpec((108, 512), lambda c, i: (0, 0)),
            pl.BlockSpec((1, 512), lambda c, i: (0, 0)),
            pl.BlockSpec((768, 256), lambda c, i: (0, 0)),
            pl.BlockSpec((1, 256), lambda c, i: (0, 0)),
            pl.BlockSpec((1280, 50), lambda c, i: (0, 0)),
            pl.BlockSpec((1, 50), lambda c, i: (0, 0)),
            pl.BlockSpec((50, n_tasks * 10), lambda c, i: (0, 0)),
            pl.BlockSpec((1, n_tasks * 10), lambda c, i: (0, 0)),
        ],
        out_specs=pl.BlockSpec((tb, n_tasks * 10), lambda c, i: (c * nh + i, 0)),
        scratch_shapes=[pltpu.VMEM((tb * 10 + 8, 144), jnp.float32)],
        compiler_params=pltpu.CompilerParams(
            dimension_semantics=("parallel", "arbitrary")),
    )(xs, w1, t1, w2, t2, wf, bf, wt, bt)
    return out.reshape(b, n_tasks, 10)
